# baseline (speedup 1.0000x reference)
import jax
import jax.numpy as jnp
from jax import lax
from jax.experimental import pallas as pl
from jax.experimental.pallas import tpu as pltpu
from jax.experimental.pallas import tpu_sc as plsc

N = 10000
D = 128
E = 320000
NG = 64
NC = 10

NCORES = 2
NSUB = 16
NW = NCORES * NSUB
EDGES_PER_TILE = E // NW
CHUNK = 40
STEPS = EDGES_PER_TILE // CHUNK
N_PAD = 10240
ROWS_PER_TILE = N_PAD // NSUB

BN = 1000
GRID = N // BN


NROWBUF = 8
NIDX = 16
PEEL = 16
LOOP_LO = PEEL
LOOP_HI = PEEL + ((STEPS - 1 - 9 - PEEL + 1) // PEEL) * PEEL


def _sc_agg_body(h_hbm, src_hbm, dst_hbm, zeros_hbm, out_hbm,
                 svs, dvs, rows, agg_sh, isems, gsems, ssem):
    cid = lax.axis_index("c")
    sid = lax.axis_index("s")
    wid = sid * NCORES + cid
    r0 = sid * ROWS_PER_TILE

    pltpu.sync_copy(zeros_hbm.at[pl.ds(r0, ROWS_PER_TILE)],
                    agg_sh.at[pl.ds(r0, ROWS_PER_TILE)])
    plsc.subcore_barrier()

    def load_idx(j, s):
        base = wid * EDGES_PER_TILE + j * CHUNK
        pltpu.async_copy(src_hbm.at[pl.ds(base, CHUNK)], svs[s], isems[s])
        pltpu.async_copy(dst_hbm.at[pl.ds(base, CHUNK)], dvs[s], isems[s])

    def wait_idx(s):
        pltpu.make_async_copy(src_hbm.at[pl.ds(0, CHUNK)], svs[s], isems[s]).wait()
        pltpu.make_async_copy(src_hbm.at[pl.ds(0, CHUNK)], dvs[s], isems[s]).wait()

    def start_gather(s, b):
        pltpu.async_copy(h_hbm.at[svs[s]], rows[b], gsems[b])

    def wait_gather(b):
        pltpu.make_async_copy(h_hbm.at[pl.ds(0, CHUNK)], rows[b], gsems[b]).wait()

    def scatter(b, s):
        pltpu.async_copy(rows[b], agg_sh.at[dvs[s]], ssem, add=True)

    def wait_scatter(b):
        pltpu.make_async_copy(rows[b], agg_sh.at[pl.ds(0, CHUNK)],
                              ssem).wait()

    def stage(j, u, do_load, do_gather, do_wait_sc):
        b = u % NROWBUF
        wait_gather(b)
        if do_wait_sc:
            wait_scatter((u + 7) % NROWBUF)
        scatter(b, u % NIDX)
        if do_load:
            load_idx(j + 9, (u + 9) % NIDX)
        if do_gather:
            wait_idx((u + 7) % NIDX)
            start_gather((u + 7) % NIDX, (u + 7) % NROWBUF)

    for t in range(9):
        load_idx(t, t)
    for t in range(7):
        wait_idx(t)
        start_gather(t, t)

    for j in range(PEEL):
        stage(j, j, True, True, j >= 1)

    def body(m, carry):
        j0 = LOOP_LO + m * PEEL
        for u in range(PEEL):
            stage(j0 + u, u, True, True, True)
        return carry

    lax.fori_loop(0, (LOOP_HI - LOOP_LO) // PEEL, body, 0)

    for j in range(LOOP_HI, STEPS):
        stage(j, j, j + 9 < STEPS, j + 7 < STEPS, True)

    wait_scatter((STEPS - 1) % NROWBUF)

    plsc.subcore_barrier()

    pltpu.sync_copy(agg_sh.at[pl.ds(r0, ROWS_PER_TILE)],
                    out_hbm.at[cid, pl.ds(r0, ROWS_PER_TILE)])


def _sc_aggregate(h, src, dst, zeros):
    return pl.kernel(
        _sc_agg_body,
        out_type=jax.ShapeDtypeStruct((NCORES, N_PAD, D), jnp.float32),
        mesh=plsc.VectorSubcoreMesh(core_axis_name="c", subcore_axis_name="s"),
        scratch_types=[
            [pltpu.VMEM((CHUNK,), jnp.int32) for _ in range(NIDX)],
            [pltpu.VMEM((CHUNK,), jnp.int32) for _ in range(NIDX)],
            [pltpu.VMEM((CHUNK, D), jnp.float32) for _ in range(NROWBUF)],
            pltpu.VMEM_SHARED((N_PAD, D), jnp.float32),
            [pltpu.SemaphoreType.DMA for _ in range(NIDX)],
            [pltpu.SemaphoreType.DMA for _ in range(NROWBUF)],
            pltpu.SemaphoreType.DMA,
        ],

    )(h, src, dst, zeros)


def _leaky(v):
    return jnp.where(v > 0, v, 0.01 * v)


def _conv_norm_body(h_ref, a0_ref, a1_ref, wa_ref, ba_ref, wb_ref, bb_ref,
                    g_ref, be_ref, out_ref, w_scr, acc_ref):
    p = pl.program_id(0)
    i = pl.program_id(1)

    @pl.when(p == 0)
    def _():
        t = h_ref[...] + a0_ref[0] + a1_ref[0]
        u = _leaky(jnp.dot(t, wa_ref[...], preferred_element_type=jnp.float32)
                   + ba_ref[...])
        v = (jnp.dot(u, wb_ref[...], preferred_element_type=jnp.float32)
             + bb_ref[...])
        w = _leaky(v)
        w_scr[pl.ds(i * BN, BN), :] = w

        @pl.when(i == 0)
        def _():
            acc_ref[...] = jnp.zeros_like(acc_ref)

        s = jnp.concatenate([jnp.sum(w, 0, keepdims=True),
                             jnp.sum(w * w, 0, keepdims=True)], axis=0)
        acc_ref[...] = acc_ref[...] + s
        out_ref[...] = w

    @pl.when(p == 1)
    def _():
        s = acc_ref[...]
        m = s[0:1, :] * (1.0 / N)
        var = s[1:2, :] * (1.0 / N) - m * m
        scale = g_ref[...] * lax.rsqrt(var + 1e-5)
        out_ref[...] = (w_scr[pl.ds(i * BN, BN), :] - m) * scale + be_ref[...]


def _conv_norm(h, agg, Wa, ba, Wb, bb, g, be):
    return pl.pallas_call(
        _conv_norm_body,
        grid=(2, GRID),
        in_specs=[
            pl.BlockSpec((BN, D), lambda p, i: (i * (1 - p), 0)),
            pl.BlockSpec((1, BN, D), lambda p, i: (0, i * (1 - p), 0)),
            pl.BlockSpec((1, BN, D), lambda p, i: (1, i * (1 - p), 0)),
            pl.BlockSpec((D, D), lambda p, i: (0, 0)),
            pl.BlockSpec((1, D), lambda p, i: (0, 0)),
            pl.BlockSpec((D, D), lambda p, i: (0, 0)),
            pl.BlockSpec((1, D), lambda p, i: (0, 0)),
            pl.BlockSpec((1, D), lambda p, i: (0, 0)),
            pl.BlockSpec((1, D), lambda p, i: (0, 0)),
        ],
        out_specs=pl.BlockSpec((BN, D), lambda p, i: (i, 0)),
        out_shape=jax.ShapeDtypeStruct((N, D), jnp.float32),
        scratch_shapes=[pltpu.VMEM((N, D), jnp.float32),
                        pltpu.VMEM((2, D), jnp.float32)],
    )(h, agg, agg, Wa, ba, Wb, bb, g, be)


def _conv_pool_body(h_ref, a0_ref, a1_ref, wa_ref, ba_ref, wb_ref, bb_ref,
                    g_ref, be_ref, batch_ref, wf1_ref, bf1_ref, wf2_ref,
                    bf2_ref, out_ref, w_scr, acc_ref, p_acc):
    p = pl.program_id(0)
    i = pl.program_id(1)

    @pl.when(p == 0)
    def _():
        t = h_ref[...] + a0_ref[0] + a1_ref[0]
        u = _leaky(jnp.dot(t, wa_ref[...], preferred_element_type=jnp.float32)
                   + ba_ref[...])
        v = (jnp.dot(u, wb_ref[...], preferred_element_type=jnp.float32)
             + bb_ref[...])
        w = _leaky(v)
        w_scr[pl.ds(i * BN, BN), :] = w

        @pl.when(i == 0)
        def _():
            acc_ref[...] = jnp.zeros_like(acc_ref)
            out_ref[...] = jnp.zeros_like(out_ref)

        s = jnp.concatenate([jnp.sum(w, 0, keepdims=True),
                             jnp.sum(w * w, 0, keepdims=True)], axis=0)
        acc_ref[...] = acc_ref[...] + s

    @pl.when(p == 1)
    def _():
        s = acc_ref[...]
        m = s[0:1, :] * (1.0 / N)
        var = s[1:2, :] * (1.0 / N) - m * m
        scale = g_ref[...] * lax.rsqrt(var + 1e-5)
        h2 = (w_scr[pl.ds(i * BN, BN), :] - m) * scale + be_ref[...]

        b = batch_ref[0]
        seg = jnp.where(
            lax.broadcasted_iota(jnp.int32, (NG, BN), 0) == b, 1.0, 0.0)

        @pl.when(i == 0)
        def _():
            p_acc[...] = jnp.zeros_like(p_acc)

        p_acc[...] = p_acc[...] + jnp.dot(seg, h2,
                                          preferred_element_type=jnp.float32)

        @pl.when(i == GRID - 1)
        def _():
            pv = p_acc[...]
            q = _leaky(jnp.dot(pv, wf1_ref[...],
                               preferred_element_type=jnp.float32)
                       + bf1_ref[...])
            z = (jnp.dot(q, wf2_ref[...], preferred_element_type=jnp.float32)
                 + bf2_ref[...])
            zmax = jnp.max(z, axis=-1, keepdims=True)
            e = jnp.exp(z - zmax)
            out_ref[...] = (z - zmax) - jnp.log(jnp.sum(e, -1, keepdims=True))


def _conv_pool(h, agg, Wa, ba, Wb, bb, g, be, batch3, Wf1, bf1, Wf2p, bf2p):
    return pl.pallas_call(
        _conv_pool_body,
        grid=(2, GRID),
        in_specs=[
            pl.BlockSpec((BN, D), lambda p, i: (i * (1 - p), 0)),
            pl.BlockSpec((1, BN, D), lambda p, i: (0, i * (1 - p), 0)),
            pl.BlockSpec((1, BN, D), lambda p, i: (1, i * (1 - p), 0)),
            pl.BlockSpec((D, D), lambda p, i: (0, 0)),
            pl.BlockSpec((1, D), lambda p, i: (0, 0)),
            pl.BlockSpec((D, D), lambda p, i: (0, 0)),
            pl.BlockSpec((1, D), lambda p, i: (0, 0)),
            pl.BlockSpec((1, D), lambda p, i: (0, 0)),
            pl.BlockSpec((1, D), lambda p, i: (0, 0)),
            pl.BlockSpec((1, 1, BN), lambda p, i: (i, 0, 0)),
            pl.BlockSpec((D, D), lambda p, i: (0, 0)),
            pl.BlockSpec((1, D), lambda p, i: (0, 0)),
            pl.BlockSpec((D, D), lambda p, i: (0, 0)),
            pl.BlockSpec((1, D), lambda p, i: (0, 0)),
        ],
        out_specs=pl.BlockSpec((NG, D), lambda p, i: (0, 0)),
        out_shape=jax.ShapeDtypeStruct((NG, D), jnp.float32),
        scratch_shapes=[pltpu.VMEM((N, D), jnp.float32),
                        pltpu.VMEM((2, D), jnp.float32),
                        pltpu.VMEM((NG, D), jnp.float32)],
    )(h, agg, agg, Wa, ba, Wb, bb, g, be, batch3, Wf1, bf1, Wf2p, bf2p)


def kernel(x, edge_index, batch, W1a, b1a, W1b, b1b, g1, be1,
           W2a, b2a, W2b, b2b, g2, be2, Wf1, bf1, Wf2, bf2):
    src = edge_index[0]
    dst = edge_index[1]
    zeros = jnp.zeros((N_PAD, D), jnp.float32)

    b1a2 = b1a.reshape(1, D)
    b1b2 = b1b.reshape(1, D)
    b2a2 = b2a.reshape(1, D)
    b2b2 = b2b.reshape(1, D)
    g1_2 = g1.reshape(1, D)
    be1_2 = be1.reshape(1, D)
    g2_2 = g2.reshape(1, D)
    be2_2 = be2.reshape(1, D)
    bf1_2 = bf1.reshape(1, D)
    Wf2p = jnp.zeros((D, D), jnp.float32).at[:, :NC].set(Wf2)
    bf2p = jnp.full((1, D), -1e30, jnp.float32).at[0, :NC].set(bf2)
    batch3 = batch.reshape(GRID, 1, BN)

    agg0 = _sc_aggregate(x, src, dst, zeros)
    h1 = _conv_norm(x, agg0, W1a, b1a2, W1b, b1b2, g1_2, be1_2)
    agg1 = _sc_aggregate(h1, src, dst, zeros)
    out = _conv_pool(h1, agg1, W2a, b2a2, W2b, b2b2, g2_2, be2_2,
                     batch3, Wf1, bf1_2, Wf2p, bf2p)
    return out[:, :NC]


# speedup vs baseline: 1.0148x; 1.0148x over previous
import jax
import jax.numpy as jnp
from jax import lax
from jax.experimental import pallas as pl
from jax.experimental.pallas import tpu as pltpu
from jax.experimental.pallas import tpu_sc as plsc

N = 10000
D = 128
E = 320000
NG = 64
NC = 10

NCORES = 2
NSUB = 16
NW = NCORES * NSUB
EDGES_PER_TILE = E // NW
CHUNK = 80
STEPS = EDGES_PER_TILE // CHUNK
N_PAD = 10240
ROWS_PER_TILE = N_PAD // NSUB

BN = 1000
GRID = N // BN


NROWBUF = 4
NIDX = 8
PEEL = 8
LOOP_LO = PEEL
LOOP_HI = PEEL + ((STEPS - 1 - 6 - PEEL + 1) // PEEL) * PEEL


def _sc_agg_body(h_hbm, src_hbm, dst_hbm, zeros_hbm, out_hbm,
                 svs, dvs, rows, agg_sh, isems, gsems, ssem):
    cid = lax.axis_index("c")
    sid = lax.axis_index("s")
    wid = sid * NCORES + cid
    r0 = sid * ROWS_PER_TILE

    pltpu.sync_copy(zeros_hbm.at[pl.ds(r0, ROWS_PER_TILE)],
                    agg_sh.at[pl.ds(r0, ROWS_PER_TILE)])
    plsc.subcore_barrier()

    def load_idx(j, s):
        base = wid * EDGES_PER_TILE + j * CHUNK
        pltpu.async_copy(src_hbm.at[pl.ds(base, CHUNK)], svs[s], isems[s])
        pltpu.async_copy(dst_hbm.at[pl.ds(base, CHUNK)], dvs[s], isems[s])

    def wait_idx(s):
        pltpu.make_async_copy(src_hbm.at[pl.ds(0, CHUNK)], svs[s], isems[s]).wait()
        pltpu.make_async_copy(src_hbm.at[pl.ds(0, CHUNK)], dvs[s], isems[s]).wait()

    def start_gather(s, b):
        pltpu.async_copy(h_hbm.at[svs[s]], rows[b], gsems[b])

    def wait_gather(b):
        pltpu.make_async_copy(h_hbm.at[pl.ds(0, CHUNK)], rows[b], gsems[b]).wait()

    def scatter(b, s):
        pltpu.async_copy(rows[b], agg_sh.at[dvs[s]], ssem, add=True)

    def wait_scatter(b):
        pltpu.make_async_copy(rows[b], agg_sh.at[pl.ds(0, CHUNK)],
                              ssem).wait()

    def stage(j, u, do_load, do_gather, do_wait_sc):
        b = u % NROWBUF
        wait_gather(b)
        if do_wait_sc:
            wait_scatter((u + 3) % NROWBUF)
        scatter(b, u % NIDX)
        if do_load:
            load_idx(j + 6, (u + 6) % NIDX)
        if do_gather:
            wait_idx((u + 3) % NIDX)
            start_gather((u + 3) % NIDX, (u + 3) % NROWBUF)

    for t in range(6):
        load_idx(t, t)
    for t in range(3):
        wait_idx(t)
        start_gather(t, t)

    for j in range(PEEL):
        stage(j, j, True, True, j >= 1)

    def body(m, carry):
        j0 = LOOP_LO + m * PEEL
        for u in range(PEEL):
            stage(j0 + u, u, True, True, True)
        return carry

    lax.fori_loop(0, (LOOP_HI - LOOP_LO) // PEEL, body, 0)

    for j in range(LOOP_HI, STEPS):
        stage(j, j, j + 6 < STEPS, j + 3 < STEPS, True)

    wait_scatter((STEPS - 1) % NROWBUF)

    plsc.subcore_barrier()

    pltpu.sync_copy(agg_sh.at[pl.ds(r0, ROWS_PER_TILE)],
                    out_hbm.at[cid, pl.ds(r0, ROWS_PER_TILE)])


def _sc_aggregate(h, src, dst, zeros):
    return pl.kernel(
        _sc_agg_body,
        out_type=jax.ShapeDtypeStruct((NCORES, N_PAD, D), jnp.float32),
        mesh=plsc.VectorSubcoreMesh(core_axis_name="c", subcore_axis_name="s"),
        scratch_types=[
            [pltpu.VMEM((CHUNK,), jnp.int32) for _ in range(NIDX)],
            [pltpu.VMEM((CHUNK,), jnp.int32) for _ in range(NIDX)],
            [pltpu.VMEM((CHUNK, D), jnp.float32) for _ in range(NROWBUF)],
            pltpu.VMEM_SHARED((N_PAD, D), jnp.float32),
            [pltpu.SemaphoreType.DMA for _ in range(NIDX)],
            [pltpu.SemaphoreType.DMA for _ in range(NROWBUF)],
            pltpu.SemaphoreType.DMA,
        ],

    )(h, src, dst, zeros)


def _leaky(v):
    return jnp.where(v > 0, v, 0.01 * v)


def _conv_norm_body(h_ref, a0_ref, a1_ref, wa_ref, ba_ref, wb_ref, bb_ref,
                    g_ref, be_ref, out_ref, w_scr, acc_ref):
    p = pl.program_id(0)
    i = pl.program_id(1)

    @pl.when(p == 0)
    def _():
        t = h_ref[...] + a0_ref[0] + a1_ref[0]
        u = _leaky(jnp.dot(t, wa_ref[...], preferred_element_type=jnp.float32)
                   + ba_ref[...])
        v = (jnp.dot(u, wb_ref[...], preferred_element_type=jnp.float32)
             + bb_ref[...])
        w = _leaky(v)
        w_scr[pl.ds(i * BN, BN), :] = w

        @pl.when(i == 0)
        def _():
            acc_ref[...] = jnp.zeros_like(acc_ref)

        s = jnp.concatenate([jnp.sum(w, 0, keepdims=True),
                             jnp.sum(w * w, 0, keepdims=True)], axis=0)
        acc_ref[...] = acc_ref[...] + s
        out_ref[...] = w

    @pl.when(p == 1)
    def _():
        s = acc_ref[...]
        m = s[0:1, :] * (1.0 / N)
        var = s[1:2, :] * (1.0 / N) - m * m
        scale = g_ref[...] * lax.rsqrt(var + 1e-5)
        out_ref[...] = (w_scr[pl.ds(i * BN, BN), :] - m) * scale + be_ref[...]


def _conv_norm(h, agg, Wa, ba, Wb, bb, g, be):
    return pl.pallas_call(
        _conv_norm_body,
        grid=(2, GRID),
        in_specs=[
            pl.BlockSpec((BN, D), lambda p, i: (i * (1 - p), 0)),
            pl.BlockSpec((1, BN, D), lambda p, i: (0, i * (1 - p), 0)),
            pl.BlockSpec((1, BN, D), lambda p, i: (1, i * (1 - p), 0)),
            pl.BlockSpec((D, D), lambda p, i: (0, 0)),
            pl.BlockSpec((1, D), lambda p, i: (0, 0)),
            pl.BlockSpec((D, D), lambda p, i: (0, 0)),
            pl.BlockSpec((1, D), lambda p, i: (0, 0)),
            pl.BlockSpec((1, D), lambda p, i: (0, 0)),
            pl.BlockSpec((1, D), lambda p, i: (0, 0)),
        ],
        out_specs=pl.BlockSpec((BN, D), lambda p, i: (i, 0)),
        out_shape=jax.ShapeDtypeStruct((N, D), jnp.float32),
        scratch_shapes=[pltpu.VMEM((N, D), jnp.float32),
                        pltpu.VMEM((2, D), jnp.float32)],
    )(h, agg, agg, Wa, ba, Wb, bb, g, be)


def _conv_pool_body(h_ref, a0_ref, a1_ref, wa_ref, ba_ref, wb_ref, bb_ref,
                    g_ref, be_ref, batch_ref, wf1_ref, bf1_ref, wf2_ref,
                    bf2_ref, out_ref, w_scr, acc_ref, p_acc):
    p = pl.program_id(0)
    i = pl.program_id(1)

    @pl.when(p == 0)
    def _():
        t = h_ref[...] + a0_ref[0] + a1_ref[0]
        u = _leaky(jnp.dot(t, wa_ref[...], preferred_element_type=jnp.float32)
                   + ba_ref[...])
        v = (jnp.dot(u, wb_ref[...], preferred_element_type=jnp.float32)
             + bb_ref[...])
        w = _leaky(v)
        w_scr[pl.ds(i * BN, BN), :] = w

        @pl.when(i == 0)
        def _():
            acc_ref[...] = jnp.zeros_like(acc_ref)
            out_ref[...] = jnp.zeros_like(out_ref)

        s = jnp.concatenate([jnp.sum(w, 0, keepdims=True),
                             jnp.sum(w * w, 0, keepdims=True)], axis=0)
        acc_ref[...] = acc_ref[...] + s

    @pl.when(p == 1)
    def _():
        s = acc_ref[...]
        m = s[0:1, :] * (1.0 / N)
        var = s[1:2, :] * (1.0 / N) - m * m
        scale = g_ref[...] * lax.rsqrt(var + 1e-5)
        h2 = (w_scr[pl.ds(i * BN, BN), :] - m) * scale + be_ref[...]

        b = batch_ref[0]
        seg = jnp.where(
            lax.broadcasted_iota(jnp.int32, (NG, BN), 0) == b, 1.0, 0.0)

        @pl.when(i == 0)
        def _():
            p_acc[...] = jnp.zeros_like(p_acc)

        p_acc[...] = p_acc[...] + jnp.dot(seg, h2,
                                          preferred_element_type=jnp.float32)

        @pl.when(i == GRID - 1)
        def _():
            pv = p_acc[...]
            q = _leaky(jnp.dot(pv, wf1_ref[...],
                               preferred_element_type=jnp.float32)
                       + bf1_ref[...])
            z = (jnp.dot(q, wf2_ref[...], preferred_element_type=jnp.float32)
                 + bf2_ref[...])
            zmax = jnp.max(z, axis=-1, keepdims=True)
            e = jnp.exp(z - zmax)
            out_ref[...] = (z - zmax) - jnp.log(jnp.sum(e, -1, keepdims=True))


def _conv_pool(h, agg, Wa, ba, Wb, bb, g, be, batch3, Wf1, bf1, Wf2p, bf2p):
    return pl.pallas_call(
        _conv_pool_body,
        grid=(2, GRID),
        in_specs=[
            pl.BlockSpec((BN, D), lambda p, i: (i * (1 - p), 0)),
            pl.BlockSpec((1, BN, D), lambda p, i: (0, i * (1 - p), 0)),
            pl.BlockSpec((1, BN, D), lambda p, i: (1, i * (1 - p), 0)),
            pl.BlockSpec((D, D), lambda p, i: (0, 0)),
            pl.BlockSpec((1, D), lambda p, i: (0, 0)),
            pl.BlockSpec((D, D), lambda p, i: (0, 0)),
            pl.BlockSpec((1, D), lambda p, i: (0, 0)),
            pl.BlockSpec((1, D), lambda p, i: (0, 0)),
            pl.BlockSpec((1, D), lambda p, i: (0, 0)),
            pl.BlockSpec((1, 1, BN), lambda p, i: (i, 0, 0)),
            pl.BlockSpec((D, D), lambda p, i: (0, 0)),
            pl.BlockSpec((1, D), lambda p, i: (0, 0)),
            pl.BlockSpec((D, D), lambda p, i: (0, 0)),
            pl.BlockSpec((1, D), lambda p, i: (0, 0)),
        ],
        out_specs=pl.BlockSpec((NG, D), lambda p, i: (0, 0)),
        out_shape=jax.ShapeDtypeStruct((NG, D), jnp.float32),
        scratch_shapes=[pltpu.VMEM((N, D), jnp.float32),
                        pltpu.VMEM((2, D), jnp.float32),
                        pltpu.VMEM((NG, D), jnp.float32)],
    )(h, agg, agg, Wa, ba, Wb, bb, g, be, batch3, Wf1, bf1, Wf2p, bf2p)


def kernel(x, edge_index, batch, W1a, b1a, W1b, b1b, g1, be1,
           W2a, b2a, W2b, b2b, g2, be2, Wf1, bf1, Wf2, bf2):
    src = edge_index[0]
    dst = edge_index[1]
    zeros = jnp.zeros((N_PAD, D), jnp.float32)

    b1a2 = b1a.reshape(1, D)
    b1b2 = b1b.reshape(1, D)
    b2a2 = b2a.reshape(1, D)
    b2b2 = b2b.reshape(1, D)
    g1_2 = g1.reshape(1, D)
    be1_2 = be1.reshape(1, D)
    g2_2 = g2.reshape(1, D)
    be2_2 = be2.reshape(1, D)
    bf1_2 = bf1.reshape(1, D)
    Wf2p = jnp.zeros((D, D), jnp.float32).at[:, :NC].set(Wf2)
    bf2p = jnp.full((1, D), -1e30, jnp.float32).at[0, :NC].set(bf2)
    batch3 = batch.reshape(GRID, 1, BN)

    agg0 = _sc_aggregate(x, src, dst, zeros)
    h1 = _conv_norm(x, agg0, W1a, b1a2, W1b, b1b2, g1_2, be1_2)
    agg1 = _sc_aggregate(h1, src, dst, zeros)
    out = _conv_pool(h1, agg1, W2a, b2a2, W2b, b2b2, g2_2, be2_2,
                     batch3, Wf1, bf1_2, Wf2p, bf2p)
    return out[:, :NC]
